# baseline (speedup 1.0000x reference)
import functools
import jax
import jax.numpy as jnp
from jax import lax
from jax.experimental import pallas as pl
from jax.experimental.pallas import tpu as pltpu
from jax.experimental.pallas import tpu_sc as plsc

EMBED_DIM = 128
NUM_EMOTIONS = 8
BATCH = 16384
BT = 8192
NB = BATCH // BT


CHUNKS = 4


def _make_sc_gather(V, D, B):
    info = plsc.get_sparse_core_info()
    NC, NS = info.num_cores, info.num_subcores
    NW = NC * NS
    b_per_w = B // NW
    c = b_per_w // CHUNKS
    mesh = plsc.VectorSubcoreMesh(core_axis_name="c", subcore_axis_name="s")

    @functools.partial(
        pl.kernel,
        mesh=mesh,
        out_type=jax.ShapeDtypeStruct((B, D), jnp.float32),
        scratch_types=[
            pltpu.VMEM((CHUNKS, c), jnp.int32),
            pltpu.VMEM((2, c, D), jnp.float32),
            pltpu.SemaphoreType.DMA,
            pltpu.SemaphoreType.DMA,
        ],
    )
    def gather_k(table_hbm, idx_hbm, out_hbm, idx_v, rows_v, gsem, wsem):
        wid = lax.axis_index("s") * NC + lax.axis_index("c")
        base = wid * b_per_w
        pltpu.sync_copy(idx_hbm.at[pl.ds(wid * CHUNKS, CHUNKS)], idx_v)

        def gather(k):
            return pltpu.async_copy(
                table_hbm.at[idx_v.at[k]], rows_v.at[k % 2], gsem)

        def write(k):
            return pltpu.async_copy(
                rows_v.at[k % 2], out_hbm.at[pl.ds(base + k * c, c)], wsem)

        gh = [gather(0)]
        wh = []
        for k in range(CHUNKS):
            gh[k].wait()
            wh.append(write(k))
            if k + 1 < CHUNKS:
                if k >= 1:
                    wh[k - 1].wait()
                gh.append(gather(k + 1))
        for h in wh[max(0, CHUNKS - 2):]:
            h.wait()

    return gather_k


def _tc_fused(spk_ref, eid_ref, etab_ref, w_ref, b_ref, out_ref):
    w1 = w_ref[:, 0:EMBED_DIM]
    w2 = w_ref[:, EMBED_DIM:2 * EMBED_DIM]
    el = lax.dot_general(
        etab_ref[...], w1, (((1,), (1,)), ((), ())),
        preferred_element_type=jnp.float32,
    ) + b_ref[...]

    eid = eid_ref[0, :, :]
    iot = lax.broadcasted_iota(jnp.int32, (NUM_EMOTIONS, BT), 0)
    oh = (iot == eid).astype(jnp.float32)
    emo_part = lax.dot_general(
        oh, el, (((0,), (0,)), ((), ())),
        preferred_element_type=jnp.float32,
    )

    spk_part = lax.dot_general(
        spk_ref[...], w2, (((1,), (1,)), ((), ())),
        preferred_element_type=jnp.float32,
    )
    out_ref[...] = spk_part + emo_part


def kernel(emotion_id, speaker_id, emotion_table, speaker_table, W, b):
    V, D = speaker_table.shape
    info = plsc.get_sparse_core_info()
    nw = info.num_cores * info.num_subcores
    c = BATCH // (nw * CHUNKS)
    sid2 = speaker_id.astype(jnp.int32).reshape(nw * CHUNKS, c)
    spk_rows = _make_sc_gather(V, D, BATCH)(speaker_table, sid2)

    eid3 = emotion_id.astype(jnp.int32).reshape(NB, 1, BT)
    b2 = b.reshape(1, EMBED_DIM)

    out = pl.pallas_call(
        _tc_fused,
        grid=(NB,),
        in_specs=[
            pl.BlockSpec((BT, EMBED_DIM), lambda i: (i, 0)),
            pl.BlockSpec((1, 1, BT), lambda i: (i, 0, 0)),
            pl.BlockSpec((NUM_EMOTIONS, EMBED_DIM), lambda i: (0, 0)),
            pl.BlockSpec((EMBED_DIM, 2 * EMBED_DIM), lambda i: (0, 0)),
            pl.BlockSpec((1, EMBED_DIM), lambda i: (0, 0)),
        ],
        out_specs=pl.BlockSpec((BT, EMBED_DIM), lambda i: (i, 0)),
        out_shape=jax.ShapeDtypeStruct((BATCH, EMBED_DIM), jnp.float32),
        compiler_params=pltpu.CompilerParams(
            dimension_semantics=("parallel",),
        ),
    )(spk_rows, eid3, emotion_table, W, b2)
    return out


# speedup vs baseline: 1.0229x; 1.0229x over previous
import functools
import jax
import jax.numpy as jnp
from jax import lax
from jax.experimental import pallas as pl
from jax.experimental.pallas import tpu as pltpu
from jax.experimental.pallas import tpu_sc as plsc

EMBED_DIM = 128
NUM_EMOTIONS = 8
BATCH = 16384
BT = 8192
NB = BATCH // BT


def _make_sc_gather(V, D, B):
    info = plsc.get_sparse_core_info()
    NC, NS = info.num_cores, info.num_subcores
    NW = NC * NS
    b_per_w = B // NW
    mesh = plsc.VectorSubcoreMesh(core_axis_name="c", subcore_axis_name="s")

    h = b_per_w // 2

    @functools.partial(
        pl.kernel,
        mesh=mesh,
        out_type=jax.ShapeDtypeStruct((B, D), jnp.float32),
        scratch_types=[
            pltpu.VMEM((h,), jnp.int32),
            pltpu.VMEM((h,), jnp.int32),
            pltpu.VMEM((h, D), jnp.float32),
            pltpu.VMEM((h, D), jnp.float32),
            pltpu.SemaphoreType.DMA,
            pltpu.SemaphoreType.DMA,
        ],
    )
    def gather_k(table_hbm, idx_hbm, out_hbm, idxa_v, idxb_v, rows_a, rows_b,
                 gsem, wsem):
        wid = lax.axis_index("s") * NC + lax.axis_index("c")
        base = wid * b_per_w
        pltpu.sync_copy(idx_hbm.at[pl.ds(base, h)], idxa_v)
        pltpu.sync_copy(idx_hbm.at[pl.ds(base + h, h)], idxb_v)
        pltpu.async_copy(table_hbm.at[idxa_v], rows_a, gsem).wait()
        wa = pltpu.async_copy(rows_a, out_hbm.at[pl.ds(base, h)], wsem)
        pltpu.async_copy(table_hbm.at[idxb_v], rows_b, gsem).wait()
        wb = pltpu.async_copy(rows_b, out_hbm.at[pl.ds(base + h, h)], wsem)
        wa.wait()
        wb.wait()

    return gather_k


def _tc_fused(spk_ref, eid_ref, etab_ref, w_ref, b_ref, out_ref):
    w1 = w_ref[:, 0:EMBED_DIM]
    w2 = w_ref[:, EMBED_DIM:2 * EMBED_DIM]
    el = lax.dot_general(
        etab_ref[...], w1, (((1,), (1,)), ((), ())),
        preferred_element_type=jnp.float32,
    ) + b_ref[...]

    eid = eid_ref[0, :, :]
    iot = lax.broadcasted_iota(jnp.int32, (NUM_EMOTIONS, BT), 0)
    oh = (iot == eid).astype(jnp.float32)
    emo_part = lax.dot_general(
        oh, el, (((0,), (0,)), ((), ())),
        preferred_element_type=jnp.float32,
    )

    spk_part = lax.dot_general(
        spk_ref[...], w2, (((1,), (1,)), ((), ())),
        preferred_element_type=jnp.float32,
    )
    out_ref[...] = spk_part + emo_part


def kernel(emotion_id, speaker_id, emotion_table, speaker_table, W, b):
    V, D = speaker_table.shape
    spk_rows = _make_sc_gather(V, D, BATCH)(
        speaker_table, speaker_id.astype(jnp.int32))

    eid3 = emotion_id.astype(jnp.int32).reshape(NB, 1, BT)
    b2 = b.reshape(1, EMBED_DIM)

    out = pl.pallas_call(
        _tc_fused,
        grid=(NB,),
        in_specs=[
            pl.BlockSpec((BT, EMBED_DIM), lambda i: (i, 0)),
            pl.BlockSpec((1, 1, BT), lambda i: (i, 0, 0)),
            pl.BlockSpec((NUM_EMOTIONS, EMBED_DIM), lambda i: (0, 0)),
            pl.BlockSpec((EMBED_DIM, 2 * EMBED_DIM), lambda i: (0, 0)),
            pl.BlockSpec((1, EMBED_DIM), lambda i: (0, 0)),
        ],
        out_specs=pl.BlockSpec((BT, EMBED_DIM), lambda i: (i, 0)),
        out_shape=jax.ShapeDtypeStruct((BATCH, EMBED_DIM), jnp.float32),
        compiler_params=pltpu.CompilerParams(
            dimension_semantics=("parallel",),
        ),
    )(spk_rows, eid3, emotion_table, W, b2)
    return out


# speedup vs baseline: 1.0579x; 1.0342x over previous
import functools
import jax
import jax.numpy as jnp
from jax import lax
from jax.experimental import pallas as pl
from jax.experimental.pallas import tpu as pltpu
from jax.experimental.pallas import tpu_sc as plsc

EMBED_DIM = 128
NUM_EMOTIONS = 8
BATCH = 16384
BT = 8192
NB = BATCH // BT


def _make_sc_gather(V, D, B):
    info = plsc.get_sparse_core_info()
    NC, NS = info.num_cores, info.num_subcores
    NW = NC * NS
    b_per_w = B // NW
    mesh = plsc.VectorSubcoreMesh(core_axis_name="c", subcore_axis_name="s")

    @functools.partial(
        pl.kernel,
        mesh=mesh,
        out_type=jax.ShapeDtypeStruct((B, D), jnp.float32),
        scratch_types=[
            pltpu.VMEM((b_per_w,), jnp.int32),
            pltpu.VMEM((b_per_w, D), jnp.float32),
            pltpu.SemaphoreType.DMA,
        ],
    )
    def gather_k(table_hbm, idx_hbm, out_hbm, idx_v, rows_v, sem):
        wid = lax.axis_index("s") * NC + lax.axis_index("c")
        base = wid * b_per_w
        pltpu.sync_copy(idx_hbm.at[pl.ds(base, b_per_w)], idx_v)
        pltpu.async_copy(table_hbm.at[idx_v], rows_v, sem).wait()
        pltpu.sync_copy(rows_v, out_hbm.at[pl.ds(base, b_per_w)])

    return gather_k


def _tc_fused(spk_ref, eid_ref, etab_ref, w_ref, b_ref, out_ref):
    w1 = w_ref[:, 0:EMBED_DIM]
    w2 = w_ref[:, EMBED_DIM:2 * EMBED_DIM]
    el = lax.dot_general(
        etab_ref[...], w1, (((1,), (1,)), ((), ())),
        preferred_element_type=jnp.float32,
    ) + b_ref[...]

    eid = eid_ref[0, :, :]
    iot = lax.broadcasted_iota(jnp.int32, (NUM_EMOTIONS, BT), 0)
    oh = (iot == eid).astype(jnp.float32)
    emo_part = lax.dot_general(
        oh, el, (((0,), (0,)), ((), ())),
        preferred_element_type=jnp.float32,
    )

    spk_part = lax.dot_general(
        spk_ref[...], w2, (((1,), (1,)), ((), ())),
        preferred_element_type=jnp.float32,
    )
    out_ref[...] = spk_part + emo_part


def kernel(emotion_id, speaker_id, emotion_table, speaker_table, W, b):
    V, D = speaker_table.shape
    spk_rows = _make_sc_gather(V, D, BATCH)(
        speaker_table, speaker_id.astype(jnp.int32))

    eid3 = emotion_id.astype(jnp.int32).reshape(NB, 1, BT)
    b2 = b.reshape(1, EMBED_DIM)

    out = pl.pallas_call(
        _tc_fused,
        grid=(NB,),
        in_specs=[
            pl.BlockSpec((BT, EMBED_DIM), lambda i: (i, 0)),
            pl.BlockSpec((1, 1, BT), lambda i: (i, 0, 0)),
            pl.BlockSpec((NUM_EMOTIONS, EMBED_DIM), lambda i: (0, 0)),
            pl.BlockSpec((EMBED_DIM, 2 * EMBED_DIM), lambda i: (0, 0)),
            pl.BlockSpec((1, EMBED_DIM), lambda i: (0, 0)),
        ],
        out_specs=pl.BlockSpec((BT, EMBED_DIM), lambda i: (i, 0)),
        out_shape=jax.ShapeDtypeStruct((BATCH, EMBED_DIM), jnp.float32),
        compiler_params=pltpu.CompilerParams(
            dimension_semantics=("parallel",),
        ),
    )(spk_rows, eid3, emotion_table, W, b2)
    return out
